# baseline (speedup 1.0000x reference)
import functools

import jax
import jax.numpy as jnp
from jax import lax
from jax.experimental import pallas as pl
from jax.experimental.pallas import tpu as pltpu
from jax.experimental.pallas import tpu_sc as plsc

_N = 128
_P = 128
_T = 10
_OFFSETS_STRIDE = 4.0
_KEY_ITEM_WEIGHT = 0.5
_IGNORE_BOUND = 1000.0
_BETA = 1.0 / _OFFSETS_STRIDE
_JC = 16

_NW = 32
_RPW = _N // _NW


def _smooth_l1(pred, target):
    diff = jnp.abs(pred - target)
    return jnp.where(diff < _BETA, 0.5 * diff * diff / _BETA, diff - 0.5 * _BETA)


def _tc_kernel(
    pxf, pyf, oxf, oyf, kxf, kyf, mf,
    gxc, gyc, gxrc, gyrc, pxc, pyc,
    out_ref, jb_ref, mv_ref,
    runmin, seltx, selty, mn2, jbest,
):
    i = pl.program_id(0)

    @pl.when(i == 0)
    def _():
        runmin[...] = jnp.full((_P, _N), jnp.inf, jnp.float32)
        mn2[...] = jnp.full((_P, _N), jnp.inf, jnp.float32)

    gxr3 = gxrc[...][:, None, :]
    gyr3 = gyrc[...][:, None, :]
    bx3 = gxc[...][:, None, :] - gxr3
    by3 = gyc[...][:, None, :] - gyr3
    c3 = bx3 * bx3 + by3 * by3
    ncr = jnp.where(c3 > 1e-30, -float(_T) / c3, 0.0)
    c100 = c3 * (1.0 / (_T * _T))
    c50 = c100 + c100
    bxT = bx3 * (1.0 / _T)
    byT = by3 * (1.0 / _T)

    px3 = pxf[...][None, :, :]
    py3 = pyf[...][None, :, :]
    dx = gxr3 - px3
    dy = gyr3 - py3
    a3 = dx * dx + dy * dy
    e3 = dx * bx3 + dy * by3
    xs = e3 * ncr
    sf = jnp.clip(jnp.floor(xs), 0.0, float(_T - 2))
    e5 = e3 * (2.0 / _T)
    d1 = (c100 * sf + e5) * sf + a3
    delta = c50 * sf + (c100 + e5)
    take1 = delta >= 0.0
    d2 = d1 + delta
    dmin = jnp.where(take1, d1, d2)
    s_at = jnp.where(take1, sf, sf + 1.0)
    tx3 = bxT * s_at + gxr3
    ty3 = byT * s_at + gyr3

    rm = runmin[...]
    sx = seltx[...]
    sy = selty[...]
    for jj in range(_JC):
        upd = dmin[jj] < rm
        rm = jnp.where(upd, dmin[jj], rm)
        sx = jnp.where(upd, tx3[jj], sx)
        sy = jnp.where(upd, ty3[jj], sy)
    runmin[...] = rm
    seltx[...] = sx
    selty[...] = sy

    kx3 = kxf[...][None, :, :]
    ky3 = kyf[...][None, :, :]
    pxr = pxc[...]
    pyr = pyc[...]
    dx2 = pxr[:, None, :] - kx3
    dy2 = pyr[:, None, :] - ky3
    dd2 = dx2 * dx2 + dy2 * dy2

    m2v = mn2[...]
    jb = jbest[...]
    for jj in range(_JC):
        upd = dd2[jj] < m2v
        m2v = jnp.where(upd, dd2[jj], m2v)
        jb = jnp.where(upd, float(jj) + float(_JC) * i.astype(jnp.float32), jb)
    mn2[...] = m2v
    jbest[...] = jb

    @pl.when(i == pl.num_programs(0) - 1)
    def _():
        inv = 1.0 / _OFFSETS_STRIDE
        bound = _IGNORE_BOUND * _IGNORE_BOUND
        valid1 = rm <= bound
        sl1 = _smooth_l1(oxf[...], (sx - pxf[...]) * inv) + _smooth_l1(
            oyf[...], (sy - pyf[...]) * inv
        )
        out_ref[0, 0] = jnp.sum(jnp.where(valid1, sl1, 0.0))
        out_ref[0, 1] = jnp.sum(valid1.astype(jnp.float32))
        valid2 = m2v <= bound
        mv_ref[...] = jnp.where(
            jnp.logical_and(mf[...] > 0.0, valid2), 1.0, 0.0
        )
        lane_n = lax.broadcasted_iota(jnp.int32, (_P, _N), 1)
        jb_ref[...] = jb.astype(jnp.int32) + lane_n * _P


def _tc_call(px, py, ox, oy, kx, ky, m, gx, gy, gxr, gyr):
    full = pl.BlockSpec((_P, _N), lambda i: (0, 0))
    chunk = pl.BlockSpec((_JC, _N), lambda i: (i, 0))
    return pl.pallas_call(
        _tc_kernel,
        grid=(_P // _JC,),
        in_specs=[full] * 7 + [chunk] * 6,
        out_specs=[
            pl.BlockSpec(memory_space=pltpu.SMEM),
            full,
            full,
        ],
        out_shape=[
            jax.ShapeDtypeStruct((1, 2), jnp.float32),
            jax.ShapeDtypeStruct((_P, _N), jnp.int32),
            jax.ShapeDtypeStruct((_P, _N), jnp.float32),
        ],
        scratch_shapes=[pltpu.VMEM((_P, _N), jnp.float32)] * 5,
    )(px, py, ox, oy, kx, ky, m, gx, gy, gxr, gyr, px, py)


_CH = _RPW * _P


def _make_sc_item2():
    return functools.partial(
        pl.kernel,
        mesh=plsc.VectorSubcoreMesh(core_axis_name="c", subcore_axis_name="s"),
        out_type=jax.ShapeDtypeStruct((_NW, 2, 16), jnp.float32),
        scratch_types=[
            pltpu.VMEM((_RPW, _P), jnp.int32),
            pltpu.VMEM((_RPW, _P), jnp.float32),
            pltpu.VMEM((_RPW, _P), jnp.float32),
            pltpu.VMEM((_RPW, _P), jnp.float32),
            pltpu.VMEM((_RPW, _P), jnp.float32),
            pltpu.VMEM((_RPW, _P), jnp.float32),
            pltpu.VMEM((_RPW, _P), jnp.float32),
            pltpu.VMEM((_RPW, _P), jnp.float32),
            pltpu.VMEM((2, 16), jnp.float32),
            pltpu.SemaphoreType.DMA,
        ],
    )(_sc_item2)


def _sc_item2(
    px_hbm, py_hbm, ox_hbm, oy_hbm, jb_hbm, kx_hbm, ky_hbm, mv_hbm, out_hbm,
    idxv, pxg, pyg, oxg, oyg, kxv, kyv, mvv, outv, sem,
):
    w = lax.axis_index("s") * 2 + lax.axis_index("c")
    base = w * _RPW
    pltpu.sync_copy(jb_hbm.at[pl.ds(base, _RPW)], idxv)
    pltpu.sync_copy(kx_hbm.at[pl.ds(base, _RPW)], kxv)
    pltpu.sync_copy(ky_hbm.at[pl.ds(base, _RPW)], kyv)
    pltpu.sync_copy(mv_hbm.at[pl.ds(base, _RPW)], mvv)

    copies = []
    for nl in range(_RPW):
        row = idxv.at[nl]
        copies.append(pltpu.async_copy(px_hbm.at[row], pxg.at[nl], sem))
        copies.append(pltpu.async_copy(py_hbm.at[row], pyg.at[nl], sem))
        copies.append(pltpu.async_copy(ox_hbm.at[row], oxg.at[nl], sem))
        copies.append(pltpu.async_copy(oy_hbm.at[row], oyg.at[nl], sem))
    for c in copies:
        c.wait()

    inv = 1.0 / _OFFSETS_STRIDE
    acc = jnp.zeros((16,), jnp.float32)
    cnt = jnp.zeros((16,), jnp.float32)
    for o in range(_RPW * _P // 16):
        nl = o // (_P // 16)
        k0 = (o % (_P // 16)) * 16
        pxl = pxg[nl, pl.ds(k0, 16)]
        pyl = pyg[nl, pl.ds(k0, 16)]
        oxl = oxg[nl, pl.ds(k0, 16)]
        oyl = oyg[nl, pl.ds(k0, 16)]
        kxl = kxv[nl, pl.ds(k0, 16)]
        kyl = kyv[nl, pl.ds(k0, 16)]
        mvl = mvv[nl, pl.ds(k0, 16)]
        sl = _smooth_l1(oxl, (kxl - pxl) * inv) + _smooth_l1(oyl, (kyl - pyl) * inv)
        acc = acc + mvl * sl
        cnt = cnt + mvl
    outv[0, :] = acc
    outv[1, :] = cnt
    pltpu.sync_copy(outv, out_hbm.at[w])


def kernel(pred_contours, pred_offsets, gt_contours, gt_key_points, gt_key_points_mask):
    px = pred_contours[..., 0].T
    py = pred_contours[..., 1].T
    ox = pred_offsets[..., 0].T
    oy = pred_offsets[..., 1].T
    gx = gt_contours[..., 0].T
    gy = gt_contours[..., 1].T
    gxr = jnp.roll(gx, 1, axis=0)
    gyr = jnp.roll(gy, 1, axis=0)
    kx = gt_key_points[..., 0].T
    ky = gt_key_points[..., 1].T
    m = gt_key_points_mask.astype(jnp.float32).T

    part1, jb, mval = _tc_call(px, py, ox, oy, kx, ky, m, gx, gy, gxr, gyr)

    flat = _N * _P
    partials = _make_sc_item2()(
        pred_contours[..., 0].reshape(flat),
        pred_contours[..., 1].reshape(flat),
        pred_offsets[..., 0].reshape(flat),
        pred_offsets[..., 1].reshape(flat),
        jb.T,
        gt_key_points[..., 0],
        gt_key_points[..., 1],
        mval.T,
    )

    s1 = part1[0, 0]
    c1 = part1[0, 1]
    s2 = jnp.sum(partials[:, 0, :])
    c2 = jnp.sum(partials[:, 1, :])
    denom1 = jnp.maximum(c1 * 2.0, 1.0)
    denom2 = jnp.maximum(c2 * 2.0, 1.0)
    return (s1 / denom1) * (1.0 - _KEY_ITEM_WEIGHT) + (s2 / denom2) * _KEY_ITEM_WEIGHT


# speedup vs baseline: 2.1417x; 2.1417x over previous
import jax
import jax.numpy as jnp
from jax.experimental import pallas as pl
from jax.experimental.pallas import tpu as pltpu

_N = 128
_P = 128
_T = 10
_OFFSETS_STRIDE = 4.0
_KEY_ITEM_WEIGHT = 0.5
_IGNORE_BOUND = 1000.0
_BETA = 1.0 / _OFFSETS_STRIDE
_JC = 8


def _smooth_l1(pred, target):
    diff = jnp.abs(pred - target)
    return jnp.where(diff < _BETA, 0.5 * diff * diff / _BETA, diff - 0.5 * _BETA)


def _dm_kernel(
    pxf, pyf, oxf, oyf, kxf, kyf, mf,
    gxc, gyc, gxrc, gyrc, pxc, pyc, oxc, oyc,
    out_ref,
    runmin, seltx, selty, mn2, pselx, psely, oselx, osely,
):
    i = pl.program_id(0)

    @pl.when(i == 0)
    def _():
        runmin[...] = jnp.full((_P, _N), jnp.inf, jnp.float32)
        mn2[...] = jnp.full((_P, _N), jnp.inf, jnp.float32)

    gxr3 = gxrc[...][:, None, :]
    gyr3 = gyrc[...][:, None, :]
    bx3 = gxc[...][:, None, :] - gxr3
    by3 = gyc[...][:, None, :] - gyr3
    c3 = bx3 * bx3 + by3 * by3
    ncr = jnp.where(c3 > 1e-30, -float(_T) / c3, 0.0)
    c100 = c3 * (1.0 / (_T * _T))
    c50 = c100 + c100
    bxT = bx3 * (1.0 / _T)
    byT = by3 * (1.0 / _T)

    px3 = pxf[...][None, :, :]
    py3 = pyf[...][None, :, :]
    dx = gxr3 - px3
    dy = gyr3 - py3
    a3 = dx * dx + dy * dy
    e3 = dx * bx3 + dy * by3

    xs = e3 * ncr
    sf = jnp.clip(jnp.floor(xs), 0.0, float(_T - 2))
    e5 = e3 * (2.0 / _T)
    d1 = (c100 * sf + e5) * sf + a3
    delta = c50 * sf + (c100 + e5)
    take1 = delta >= 0.0
    d2 = d1 + delta
    dmin = jnp.where(take1, d1, d2)
    s_at = jnp.where(take1, sf, sf + 1.0)
    tx3 = bxT * s_at + gxr3
    ty3 = byT * s_at + gyr3

    rm = runmin[...]
    sx = seltx[...]
    sy = selty[...]
    for jj in range(_JC):
        upd = dmin[jj] < rm
        rm = jnp.where(upd, dmin[jj], rm)
        sx = jnp.where(upd, tx3[jj], sx)
        sy = jnp.where(upd, ty3[jj], sy)
    runmin[...] = rm
    seltx[...] = sx
    selty[...] = sy

    kx3 = kxf[...][None, :, :]
    ky3 = kyf[...][None, :, :]
    pxr = pxc[...]
    pyr = pyc[...]
    oxr = oxc[...]
    oyr = oyc[...]
    dx2 = pxr[:, None, :] - kx3
    dy2 = pyr[:, None, :] - ky3
    dd2 = dx2 * dx2 + dy2 * dy2

    m2v = mn2[...]
    qx = pselx[...]
    qy = psely[...]
    rx = oselx[...]
    ry = osely[...]
    for jj in range(_JC):
        upd = dd2[jj] < m2v
        m2v = jnp.where(upd, dd2[jj], m2v)
        qx = jnp.where(upd, pxr[jj][None, :], qx)
        qy = jnp.where(upd, pyr[jj][None, :], qy)
        rx = jnp.where(upd, oxr[jj][None, :], rx)
        ry = jnp.where(upd, oyr[jj][None, :], ry)
    mn2[...] = m2v
    pselx[...] = qx
    psely[...] = qy
    oselx[...] = rx
    osely[...] = ry

    @pl.when(i == pl.num_programs(0) - 1)
    def _():
        inv = 1.0 / _OFFSETS_STRIDE
        bound = _IGNORE_BOUND * _IGNORE_BOUND
        valid1 = rm <= bound
        sl1 = _smooth_l1(oxf[...], (sx - pxf[...]) * inv) + _smooth_l1(
            oyf[...], (sy - pyf[...]) * inv
        )
        s1 = jnp.sum(jnp.where(valid1, sl1, 0.0))
        c1 = jnp.sum(valid1.astype(jnp.float32))

        valid2 = m2v <= bound
        mk = jnp.logical_and(mf[...] > 0.0, valid2)
        sl2 = _smooth_l1(rx, (kxf[...] - qx) * inv) + _smooth_l1(
            ry, (kyf[...] - qy) * inv
        )
        s2s = jnp.sum(jnp.where(mk, sl2, 0.0))
        c2 = jnp.sum(mk.astype(jnp.float32))

        denom1 = jnp.maximum(c1 * 2.0, 1.0)
        denom2 = jnp.maximum(c2 * 2.0, 1.0)
        out_ref[0, 0] = (s1 / denom1) * (1.0 - _KEY_ITEM_WEIGHT) + (
            s2s / denom2
        ) * _KEY_ITEM_WEIGHT


def kernel(pred_contours, pred_offsets, gt_contours, gt_key_points, gt_key_points_mask):
    px = pred_contours[..., 0].T
    py = pred_contours[..., 1].T
    ox = pred_offsets[..., 0].T
    oy = pred_offsets[..., 1].T
    gx = gt_contours[..., 0].T
    gy = gt_contours[..., 1].T
    gxr = jnp.roll(gx, 1, axis=0)
    gyr = jnp.roll(gy, 1, axis=0)
    kx = gt_key_points[..., 0].T
    ky = gt_key_points[..., 1].T
    m = gt_key_points_mask.astype(jnp.float32).T

    full = pl.BlockSpec((_P, _N), lambda i: (0, 0))
    chunk = pl.BlockSpec((_JC, _N), lambda i: (i, 0))
    out = pl.pallas_call(
        _dm_kernel,
        grid=(_P // _JC,),
        in_specs=[full] * 7 + [chunk] * 8,
        out_specs=pl.BlockSpec(memory_space=pltpu.SMEM),
        out_shape=jax.ShapeDtypeStruct((1, 1), jnp.float32),
        scratch_shapes=[pltpu.VMEM((_P, _N), jnp.float32)] * 8,
    )(px, py, ox, oy, kx, ky, m, gx, gy, gxr, gyr, px, py, ox, oy)
    return out[0, 0]
